# baseline (speedup 1.0000x reference)
import functools
import math

import numpy as np
import jax
import jax.numpy as jnp
from jax import lax
from jax.experimental import pallas as pl
from jax.experimental.pallas import tpu as pltpu
from jax.experimental.pallas import tpu_sc as plsc

SAMPLE_RATE = 16000
SOUND_SPEED = 343.0
MAX_ORDER = 15
RIR_LENGTH = 3968
WINDOW_LENGTH = 81
PAD = WINDOW_LENGTH // 2
B = 32
M = 5
OFF = 64
LH = 1920
QH = LH // 128
QO = QH - 1


def _build_constants():
    ind = np.arange(-MAX_ORDER, MAX_ORDER + 1)
    X, Y, Z = np.meshgrid(ind, ind, ind, indexing="ij")
    xyz = np.stack([X.ravel(), Y.ravel(), Z.ravel()], axis=-1)
    xyz = xyz[np.abs(xyz).sum(axis=-1) <= MAX_ORDER].astype(np.float32)
    n = xyz.shape[0]
    exp_lo = np.abs(np.floor(xyz / 2.0)).astype(np.float32)
    exp_hi = np.abs(np.floor((xyz + 1) / 2.0)).astype(np.float32)
    odd = np.mod(xyz, 2.0) == 1.0
    xa = np.where(odd, xyz + 1.0, xyz).astype(np.float32)
    sgn = np.where(odd, -1.0, 1.0).astype(np.float32)

    n_pad = 4992
    def padT(a, fill):
        out = np.full((3, n_pad), fill, np.float32)
        out[:, :n] = a.T
        return out

    xa_p = padT(xa, 1e6)
    sgn_p = padT(sgn, 1.0)
    elo_p = padT(exp_lo, 0.0)
    ehi_p = padT(exp_hi, 0.0)

    def h_exact(j, f):
        k = j - PAD
        x = k + f
        hann = np.where(np.abs(x) <= PAD,
                        0.5 * (1 + np.cos(2 * math.pi * x / (2 * PAD))), 0.0)
        return np.sinc(x) * hann

    import numpy.polynomial.chebyshev as cheb
    nodes = (np.cos(np.pi * (np.arange(4 * M) + 0.5) / (4 * M)) + 1) / 2
    V = cheb.chebvander(2 * nodes - 1, M - 1)
    C = np.zeros((WINDOW_LENGTH, M))
    for j in range(WINDOW_LENGTH):
        c, *_ = np.linalg.lstsq(V, h_exact(j, nodes), rcond=None)
        C[j] = c

    WA = np.zeros((M, 128, 128), np.float32)
    WB = np.zeros((M, 128, 128), np.float32)
    for m in range(M):
        for cc in range(128):
            for s in range(24, 105):
                r = cc + s
                j = 104 - s
                if r < 128:
                    WA[m, r, cc] = C[j, m]
                else:
                    WB[m, r - 128, cc] = C[j, m]
    return n, n_pad, xa_p, sgn_p, elo_p, ehi_p, WA, WB


N, N_PAD, _XA, _SGN, _ELO, _EHI, _WA, _WB = _build_constants()


def _tc1_body(inp_ref, xa_ref, sgn_ref, elo_ref, ehi_ref,
              d_ref, g_ref, on_ref):
    inp = inp_ref[...]
    dist2 = None
    latt = None
    od2 = None
    for a in range(3):
        room = inp[:, a:a + 1] + 1.0
        mic = inp[:, 3 + a:4 + a] * room
        src = inp[:, 6 + a:7 + a] * room
        img = room * xa_ref[a:a + 1, :] + sgn_ref[a:a + 1, :] * src
        v = img - mic
        dist2 = v * v if a == 0 else dist2 + v * v
        if a == 0:
            tl = 1.0 - (inp[:, 9:10] * 0.84 + 0.01)
            th = tl
        elif a == 1:
            tl = 1.0 - (inp[:, 9:10] * 0.84 + 0.01)
            th = tl
        else:
            tl = 1.0 - (inp[:, 10:11] * 0.84 + 0.01)
            th = 1.0 - (inp[:, 11:12] * 0.84 + 0.01)
        term = elo_ref[a:a + 1, :] * jnp.log(tl) + ehi_ref[a:a + 1, :] * jnp.log(th)
        latt = term if a == 0 else latt + term
        ds = mic - src
        od2 = ds * ds if a == 0 else od2 + ds * ds
    dist = jnp.sqrt(dist2)
    att = jnp.exp(latt)
    amp = att / dist
    delay = dist * np.float32(SAMPLE_RATE) / np.float32(SOUND_SPEED)
    di = jnp.ceil(delay)
    f = di - delay
    u = f + f - 1.0
    nmask = lax.broadcasted_iota(jnp.int32, (1, N_PAD), 1) < N
    didx = jnp.minimum(di.astype(jnp.int32) + OFF, LH - 1)
    d_ref[...] = jnp.where(nmask, didx, 0)
    g_ref[:, pl.ds(0, N_PAD)] = jnp.where(nmask, amp, 0.0)
    tm2 = jnp.ones_like(u)
    tm1 = u
    g_ref[:, pl.ds(N_PAD, N_PAD)] = jnp.where(nmask, amp * u, 0.0)
    for m in range(2, M):
        t = 2.0 * u * tm1 - tm2
        g_ref[:, pl.ds(m * N_PAD, N_PAD)] = jnp.where(nmask, amp * t, 0.0)
        tm2, tm1 = tm1, t
    on_ref[...] = np.float32(SAMPLE_RATE) * jnp.sqrt(od2) / np.float32(SOUND_SPEED)


_tc1 = pl.pallas_call(
    _tc1_body,
    out_shape=[
        jax.ShapeDtypeStruct((B, N_PAD), jnp.int32),
        jax.ShapeDtypeStruct((B, M * N_PAD), jnp.float32),
        jax.ShapeDtypeStruct((B, 1), jnp.float32),
    ],
)


def _sc_hist_body(d_hbm, g_hbm, out_hbm, idx_v, g_v, hist_v, sem):
    wid = lax.axis_index("s") * 2 + lax.axis_index("c")
    gcp = pltpu.make_async_copy(g_hbm.at[wid], g_v, sem)
    gcp.start()
    pltpu.sync_copy(d_hbm.at[wid], idx_v)
    zero16 = jnp.zeros((16,), jnp.float32)

    def _zero(i, _):
        for k in range(8):
            hist_v[pl.ds(i * 128 + k * 16, 16)] = zero16
        return 0
    lax.fori_loop(0, (M * LH) // 128, _zero, 0)
    gcp.wait()

    def _scatter(i, _):
        for un in range(4):
            d16 = idx_v[pl.ds(i * 64 + un * 16, 16)]
            for m in range(M):
                g16 = g_v[pl.ds(m * N_PAD + i * 64 + un * 16, 16)]
                plsc.addupdate_scatter(hist_v, [d16 + (m * LH)], g16)
        return 0
    lax.fori_loop(0, N_PAD // 64, _scatter, 0)
    for m in range(M):
        pltpu.sync_copy(hist_v.at[pl.ds(m * LH, LH)],
                        out_hbm.at[pl.ds(m * (B * LH) + wid * LH, LH)])


@functools.cache
def _get_sc_hist():
    return pl.kernel(
        _sc_hist_body,
        out_type=jax.ShapeDtypeStruct((M * B * LH,), jnp.float32),
        mesh=plsc.VectorSubcoreMesh(core_axis_name="c", subcore_axis_name="s"),
        compiler_params=pltpu.CompilerParams(needs_layout_passes=False),
        scratch_types=[
            pltpu.VMEM((N_PAD,), jnp.int32),
            pltpu.VMEM((M * N_PAD,), jnp.float32),
            pltpu.VMEM((M * LH,), jnp.float32),
            pltpu.SemaphoreType.DMA,
        ],
    )


def _tc3_body(h_ref, wa_ref, wb_ref, o_ref):
    y = jnp.zeros((B * QH, 128), jnp.float32)
    z = jnp.zeros((B * QH, 128), jnp.float32)
    for m in range(M):
        h = h_ref[m]
        y = y + jnp.dot(h, wa_ref[m], preferred_element_type=jnp.float32,
                        precision=lax.Precision.HIGHEST)
        z = z + jnp.dot(h, wb_ref[m], preferred_element_type=jnp.float32,
                        precision=lax.Precision.HIGHEST)
    y3 = y.reshape(B, QH, 128)
    z3 = z.reshape(B, QH, 128)
    for q in range(QO):
        o_ref[:, pl.ds(q * 128, 128)] = y3[:, q, :] + z3[:, q + 1, :]
    o_ref[:, pl.ds(QO * 128, RIR_LENGTH - QO * 128)] = jnp.zeros(
        (B, RIR_LENGTH - QO * 128), jnp.float32)


_tc3 = pl.pallas_call(
    _tc3_body,
    out_shape=jax.ShapeDtypeStruct((B, RIR_LENGTH), jnp.float32),
)


def kernel(input):
    xa = jnp.asarray(_XA)
    sgn = jnp.asarray(_SGN)
    elo = jnp.asarray(_ELO)
    ehi = jnp.asarray(_EHI)
    d_idx, g, onset = _tc1(input, xa, sgn, elo, ehi)
    hist = _get_sc_hist()(d_idx, g)
    histr = hist.reshape(M, B * QH, 128)
    out = _tc3(histr, jnp.asarray(_WA), jnp.asarray(_WB))
    return out, onset.reshape(B)


# speedup vs baseline: 1.0389x; 1.0389x over previous
import functools
import math

import numpy as np
import jax
import jax.numpy as jnp
from jax import lax
from jax.experimental import pallas as pl
from jax.experimental.pallas import tpu as pltpu
from jax.experimental.pallas import tpu_sc as plsc

SAMPLE_RATE = 16000
SOUND_SPEED = 343.0
MAX_ORDER = 15
RIR_LENGTH = 3968
WINDOW_LENGTH = 81
PAD = WINDOW_LENGTH // 2
B = 32
M = 5
OFF = 64
LH = 1920
QH = LH // 128
QO = QH - 1


def _build_constants():
    ind = np.arange(-MAX_ORDER, MAX_ORDER + 1)
    X, Y, Z = np.meshgrid(ind, ind, ind, indexing="ij")
    xyz = np.stack([X.ravel(), Y.ravel(), Z.ravel()], axis=-1)
    xyz = xyz[np.abs(xyz).sum(axis=-1) <= MAX_ORDER].astype(np.float32)
    n = xyz.shape[0]
    exp_lo = np.abs(np.floor(xyz / 2.0)).astype(np.float32)
    exp_hi = np.abs(np.floor((xyz + 1) / 2.0)).astype(np.float32)
    odd = np.mod(xyz, 2.0) == 1.0
    xa = np.where(odd, xyz + 1.0, xyz).astype(np.float32)
    sgn = np.where(odd, -1.0, 1.0).astype(np.float32)

    n_pad = 4992
    perm = np.random.default_rng(0).permutation(n)

    def padT(a, fill):
        out = np.full((3, n_pad), fill, np.float32)
        out[:, :n] = a.T[:, perm]
        return out

    xa_p = padT(xa, 1e6)
    sgn_p = padT(sgn, 1.0)
    elo_p = padT(exp_lo, 0.0)
    ehi_p = padT(exp_hi, 0.0)
    tbl = np.concatenate(
        [xa_p, sgn_p, elo_p, ehi_p, np.zeros((4, n_pad), np.float32)], axis=0)

    def h_exact(j, f):
        k = j - PAD
        x = k + f
        hann = np.where(np.abs(x) <= PAD,
                        0.5 * (1 + np.cos(2 * math.pi * x / (2 * PAD))), 0.0)
        return np.sinc(x) * hann

    import numpy.polynomial.chebyshev as cheb
    nodes = (np.cos(np.pi * (np.arange(4 * M) + 0.5) / (4 * M)) + 1) / 2
    V = cheb.chebvander(2 * nodes - 1, M - 1)
    C = np.zeros((WINDOW_LENGTH, M))
    for j in range(WINDOW_LENGTH):
        c, *_ = np.linalg.lstsq(V, h_exact(j, nodes), rcond=None)
        C[j] = c

    WA = np.zeros((M, 128, 128), np.float32)
    WB = np.zeros((M, 128, 128), np.float32)
    for m in range(M):
        for cc in range(128):
            for s in range(24, 105):
                r = cc + s
                j = 104 - s
                if r < 128:
                    WA[m, r, cc] = C[j, m]
                else:
                    WB[m, r - 128, cc] = C[j, m]
    def split_bf16(w):
        hi = w.astype(np.float32).astype(jnp.bfloat16)
        lo = (w - np.asarray(hi, np.float32)).astype(jnp.bfloat16)
        return np.asarray(hi), np.asarray(lo)

    wa_hi, wa_lo = split_bf16(WA)
    wb_hi, wb_lo = split_bf16(WB)
    return n, n_pad, tbl, wa_hi, wa_lo, wb_hi, wb_lo


N, N_PAD, _TBL, _WAH, _WAL, _WBH, _WBL = _build_constants()


def _tc1_body(inp_ref, tbl_ref, d_ref, g_ref, on_ref):
    inp = inp_ref[...]
    dist2 = None
    latt = None
    od2 = None
    for a in range(3):
        room = inp[:, a:a + 1] + 1.0
        mic = inp[:, 3 + a:4 + a] * room
        src = inp[:, 6 + a:7 + a] * room
        img = room * tbl_ref[a:a + 1, :] + tbl_ref[3 + a:4 + a, :] * src
        v = img - mic
        dist2 = v * v if a == 0 else dist2 + v * v
        if a == 0:
            tl = 1.0 - (inp[:, 9:10] * 0.84 + 0.01)
            th = tl
        elif a == 1:
            tl = 1.0 - (inp[:, 9:10] * 0.84 + 0.01)
            th = tl
        else:
            tl = 1.0 - (inp[:, 10:11] * 0.84 + 0.01)
            th = 1.0 - (inp[:, 11:12] * 0.84 + 0.01)
        term = (tbl_ref[6 + a:7 + a, :] * jnp.log(tl)
                + tbl_ref[9 + a:10 + a, :] * jnp.log(th))
        latt = term if a == 0 else latt + term
        ds = mic - src
        od2 = ds * ds if a == 0 else od2 + ds * ds
    dist = jnp.sqrt(dist2)
    att = jnp.exp(latt)
    amp = att / dist
    delay = dist * np.float32(SAMPLE_RATE) / np.float32(SOUND_SPEED)
    di = jnp.ceil(delay)
    f = di - delay
    u = f + f - 1.0
    nmask = lax.broadcasted_iota(jnp.int32, (1, N_PAD), 1) < N
    didx = jnp.minimum(di.astype(jnp.int32) + OFF, LH - 1)
    d_ref[...] = jnp.where(nmask, didx, 0)
    g_ref[:, pl.ds(0, N_PAD)] = jnp.where(nmask, amp, 0.0)
    tm2 = jnp.ones_like(u)
    tm1 = u
    g_ref[:, pl.ds(N_PAD, N_PAD)] = jnp.where(nmask, amp * u, 0.0)
    for m in range(2, M):
        t = 2.0 * u * tm1 - tm2
        g_ref[:, pl.ds(m * N_PAD, N_PAD)] = jnp.where(nmask, amp * t, 0.0)
        tm2, tm1 = tm1, t
    on_ref[...] = np.float32(SAMPLE_RATE) * jnp.sqrt(od2) / np.float32(SOUND_SPEED)


_tc1 = pl.pallas_call(
    _tc1_body,
    out_shape=[
        jax.ShapeDtypeStruct((B, N_PAD), jnp.int32),
        jax.ShapeDtypeStruct((B, M * N_PAD), jnp.float32),
        jax.ShapeDtypeStruct((B, 1), jnp.float32),
    ],
)


def _sc_hist_body(d_hbm, g_hbm, out_hbm, idx_v, g_v, hist_v, sem):
    wid = lax.axis_index("s") * 2 + lax.axis_index("c")
    gcp = pltpu.make_async_copy(g_hbm.at[wid], g_v, sem)
    gcp.start()
    pltpu.sync_copy(d_hbm.at[wid], idx_v)
    zero16 = jnp.zeros((16,), jnp.float32)

    def _zero(i, _):
        for k in range(8):
            hist_v[pl.ds(i * 128 + k * 16, 16)] = zero16
        return 0
    lax.fori_loop(0, (M * LH) // 128, _zero, 0)
    gcp.wait()

    def _scatter(i, _):
        for un in range(4):
            d16 = idx_v[pl.ds(i * 64 + un * 16, 16)]
            for m in range(M):
                g16 = g_v[pl.ds(m * N_PAD + i * 64 + un * 16, 16)]
                plsc.addupdate_scatter(hist_v, [d16 + (m * LH)], g16)
        return 0
    lax.fori_loop(0, N_PAD // 64, _scatter, 0)
    for m in range(M):
        pltpu.sync_copy(hist_v.at[pl.ds(m * LH, LH)],
                        out_hbm.at[pl.ds(m * (B * LH) + wid * LH, LH)])


@functools.cache
def _get_sc_hist():
    return pl.kernel(
        _sc_hist_body,
        out_type=jax.ShapeDtypeStruct((M * B * LH,), jnp.float32),
        mesh=plsc.VectorSubcoreMesh(core_axis_name="c", subcore_axis_name="s"),
        compiler_params=pltpu.CompilerParams(needs_layout_passes=False),
        scratch_types=[
            pltpu.VMEM((N_PAD,), jnp.int32),
            pltpu.VMEM((M * N_PAD,), jnp.float32),
            pltpu.VMEM((M * LH,), jnp.float32),
            pltpu.SemaphoreType.DMA,
        ],
    )


def _tc3_body(h_ref, wah_ref, wal_ref, wbh_ref, wbl_ref, o_ref):
    y = jnp.zeros((B * QH, 128), jnp.float32)
    z = jnp.zeros((B * QH, 128), jnp.float32)
    for m in range(M):
        h = h_ref[m]
        hh = h.astype(jnp.bfloat16)
        hl = (h - hh.astype(jnp.float32)).astype(jnp.bfloat16)
        for hv, wa, wb in ((hh, wah_ref[m], wbh_ref[m]),
                           (hh, wal_ref[m], wbl_ref[m]),
                           (hl, wah_ref[m], wbh_ref[m])):
            y = y + jnp.dot(hv, wa, preferred_element_type=jnp.float32)
            z = z + jnp.dot(hv, wb, preferred_element_type=jnp.float32)
    y3 = y.reshape(B, QH, 128)
    z3 = z.reshape(B, QH, 128)
    for q in range(QO):
        o_ref[:, pl.ds(q * 128, 128)] = y3[:, q, :] + z3[:, q + 1, :]
    o_ref[:, pl.ds(QO * 128, RIR_LENGTH - QO * 128)] = jnp.zeros(
        (B, RIR_LENGTH - QO * 128), jnp.float32)


_tc3 = pl.pallas_call(
    _tc3_body,
    out_shape=jax.ShapeDtypeStruct((B, RIR_LENGTH), jnp.float32),
)


def kernel(input):
    d_idx, g, onset = _tc1(input, jnp.asarray(_TBL))
    hist = _get_sc_hist()(d_idx, g)
    histr = hist.reshape(M, B * QH, 128)
    out = _tc3(histr, jnp.asarray(_WAH), jnp.asarray(_WAL),
               jnp.asarray(_WBH), jnp.asarray(_WBL))
    return out, onset.reshape(B)
